# Initial kernel scaffold; baseline (speedup 1.0000x reference)
import jax
import jax.numpy as jnp
from jax.experimental import pallas as pl


def kernel(x, edge_attr, edge_index, batch, W1, b1, W2, b2, W3, b3, Wc, bc):
    raise NotImplementedError("write your pallas kernel here")


# speedup vs baseline: 27.8881x; 27.8881x over previous
import functools

import jax
import jax.numpy as jnp
from jax import lax
from jax.experimental import pallas as pl
from jax.experimental.pallas import tpu as pltpu
from jax.experimental.pallas import tpu_sc as plsc

_N = 10000
_D = 128
_G = 64
_NPAD = 10240
_E = 320000
_ETOT = _E + _N
_B = 128
_NC, _NS = 2, 16
_NW = _NC * _NS
_NBLK = -(-_ETOT // (_B * _NW))
_EPAD = _NBLK * _NW * _B
_STRIPE = _NPAD // _NS
_RB = _NPAD // 1024


def _deg_body(ib, z1, out, ibv, onesv, acc, sdma):
    c = lax.axis_index("c")
    s = lax.axis_index("s")
    w = s * _NC + c
    pltpu.sync_copy(z1.at[pl.ds(s * _STRIPE, _STRIPE)],
                    acc.at[pl.ds(s * _STRIPE, _STRIPE)])
    pltpu.sync_copy(ib.at[w], ibv)
    for g in range(_B // 16):
        onesv[pl.ds(g * 16, 16)] = jnp.ones((16,), jnp.float32)
    plsc.subcore_barrier()

    def step(i, carry):
        pltpu.async_copy(onesv, acc.at[ibv.at[i, 1]], sdma, add=True)
        return carry

    lax.fori_loop(0, _NBLK, step, 0)

    def drain(i, carry):
        pltpu.make_async_copy(onesv, acc.at[ibv.at[i, 1]], sdma).wait()
        return carry

    lax.fori_loop(0, _NBLK, drain, 0)
    plsc.subcore_barrier()
    pltpu.sync_copy(acc.at[pl.ds(s * _STRIPE, _STRIPE)],
                    out.at[pl.ds(c * _NPAD + s * _STRIPE, _STRIPE)])


@functools.cache
def _deg_call():
    mesh = plsc.VectorSubcoreMesh(core_axis_name="c", subcore_axis_name="s",
                                  num_cores=_NC, num_subcores=_NS)
    return pl.kernel(
        _deg_body,
        out_type=jax.ShapeDtypeStruct((_NC * _NPAD,), jnp.float32),
        mesh=mesh,
        scratch_types=[
            pltpu.VMEM((_NBLK, 2, _B), jnp.int32),
            pltpu.VMEM((_B,), jnp.float32),
            pltpu.VMEM_SHARED((_NPAD,), jnp.float32),
            pltpu.SemaphoreType.DMA,
        ],
    )


def _prop_body(q, ib, z2, out, ib0, ib1, ib2, rows0, rows1, acc,
               is0, is1, is2, sem0, sem1):
    c = lax.axis_index("c")
    s = lax.axis_index("s")
    w = s * _NC + c

    def start_i(j, buf, sem):
        pltpu.async_copy(ib.at[w, j], buf, sem)

    def wait_i(buf, sem):
        pltpu.make_async_copy(ib.at[w, 0], buf, sem).wait()

    def start_g(ibuf, buf, sem):
        pltpu.async_copy(q.at[ibuf.at[0]], buf, sem)

    def wait_g(buf, sem):
        pltpu.make_async_copy(q.at[pl.ds(0, _B)], buf, sem).wait()

    def scat(ibuf, buf):
        pltpu.sync_copy(buf, acc.at[ibuf.at[1]], add=True)

    s0, s1, s2 = (ib0, is0), (ib1, is1), (ib2, is2)

    def sub(blk, sA, sB, sC, tail):
        ibA, isA = sA
        ibB, isB = sB
        ibC, isC = sC
        wait_i(ibB, isB)
        start_g(ibB, rows1, sem1)
        wait_g(rows0, sem0)
        scat(ibA, rows0)
        if not tail:
            start_i(blk + 3, ibA, isA)
        wait_i(ibC, isC)
        start_g(ibC, rows0, sem0)
        wait_g(rows1, sem1)
        scat(ibB, rows1)
        if not tail:
            start_i(blk + 4, ibB, isB)

    start_i(0, ib0, is0)
    start_i(1, ib1, is1)
    start_i(2, ib2, is2)
    pltpu.sync_copy(z2, acc.at[pl.ds(s * _STRIPE, _STRIPE)])
    wait_i(ib0, is0)
    start_g(ib0, rows0, sem0)
    plsc.subcore_barrier()

    def step(k, carry):
        b = 6 * k
        sub(b, s0, s1, s2, False)
        sub(b + 2, s2, s0, s1, False)
        sub(b + 4, s1, s2, s0, False)
        return carry

    lax.fori_loop(0, (_NBLK - 3) // 6, step, 0)
    sub(_NBLK - 3, s0, s1, s2, True)
    wait_g(rows0, sem0)
    scat(ib2, rows0)
    plsc.subcore_barrier()
    pltpu.sync_copy(acc.at[pl.ds(s * _STRIPE, _STRIPE)],
                    out.at[c].at[pl.ds(s * _STRIPE, _STRIPE)])


@functools.cache
def _prop_call():
    mesh = plsc.VectorSubcoreMesh(core_axis_name="c", subcore_axis_name="s",
                                  num_cores=_NC, num_subcores=_NS)
    return pl.kernel(
        _prop_body,
        out_type=jax.ShapeDtypeStruct((_NC, _NPAD, _D), jnp.float32),
        mesh=mesh,
        scratch_types=[
            pltpu.VMEM((2, _B), jnp.int32),
            pltpu.VMEM((2, _B), jnp.int32),
            pltpu.VMEM((2, _B), jnp.int32),
            pltpu.VMEM((_B, _D), jnp.float32),
            pltpu.VMEM((_B, _D), jnp.float32),
            pltpu.VMEM_SHARED((_NPAD, _D), jnp.float32),
            pltpu.SemaphoreType.DMA,
            pltpu.SemaphoreType.DMA,
            pltpu.SemaphoreType.DMA,
            pltpu.SemaphoreType.DMA,
            pltpu.SemaphoreType.DMA,
        ],
    )


def _mm1_body(x_ref, w_ref, dg_ref, q_ref, dv_ref):
    d = dg_ref[0] + dg_ref[1]
    dinv = jnp.where(d > 0, lax.rsqrt(d), 0.0)
    dv_ref[...] = dinv
    q_ref[...] = dinv * jnp.dot(x_ref[...], w_ref[...],
                                preferred_element_type=jnp.float32)


_mm1 = pl.pallas_call(
    _mm1_body,
    grid=(_RB,),
    in_specs=[
        pl.BlockSpec((1024, _D), lambda i: (i, 0)),
        pl.BlockSpec((_D, _D), lambda i: (0, 0)),
        pl.BlockSpec((_NC, 1024, 1), lambda i: (0, i, 0)),
    ],
    out_specs=[
        pl.BlockSpec((1024, _D), lambda i: (i, 0)),
        pl.BlockSpec((1024, 1), lambda i: (i, 0)),
    ],
    out_shape=[
        jax.ShapeDtypeStruct((_NPAD, _D), jnp.float32),
        jax.ShapeDtypeStruct((_NPAD, 1), jnp.float32),
    ],
)


def _mid_body(sp_ref, dv_ref, b_ref, w_ref, q_ref):
    dinv = dv_ref[...]
    h = jnp.maximum(dinv * (sp_ref[0] + sp_ref[1]) + b_ref[...], 0.0)
    q_ref[...] = dinv * jnp.dot(h, w_ref[...],
                                preferred_element_type=jnp.float32)


_mid = pl.pallas_call(
    _mid_body,
    grid=(_RB,),
    in_specs=[
        pl.BlockSpec((_NC, 1024, _D), lambda i: (0, i, 0)),
        pl.BlockSpec((1024, 1), lambda i: (i, 0)),
        pl.BlockSpec((1, _D), lambda i: (0, 0)),
        pl.BlockSpec((_D, _D), lambda i: (0, 0)),
    ],
    out_specs=pl.BlockSpec((1024, _D), lambda i: (i, 0)),
    out_shape=jax.ShapeDtypeStruct((_NPAD, _D), jnp.float32),
)


def _fin_body(sp_ref, dv_ref, b_ref, bm_ref, wc_ref, bc_ref, out_ref,
              pool_ref):
    i = pl.program_id(0)
    dinv = dv_ref[...]
    h = jnp.maximum(dinv * (sp_ref[0] + sp_ref[1]) + b_ref[...], 0.0)
    gids = lax.broadcasted_iota(jnp.int32, (_G, 1024), 0)
    m = (gids == bm_ref[...]).astype(jnp.float32)
    part = jnp.dot(m, h, preferred_element_type=jnp.float32)

    @pl.when(i == 0)
    def _():
        pool_ref[...] = part

    @pl.when(i > 0)
    def _():
        pool_ref[...] += part

    @pl.when(i == _RB - 1)
    def _():
        logits = jnp.dot(pool_ref[...], wc_ref[...],
                         preferred_element_type=jnp.float32) + bc_ref[...]
        col = lax.broadcasted_iota(jnp.int32, (_G, _D), 1)
        valid = col < 10
        neg = jnp.where(valid, logits, -jnp.inf)
        mx = jnp.max(neg, axis=1, keepdims=True)
        ez = jnp.where(valid, jnp.exp(logits - mx), 0.0)
        lse = jnp.log(jnp.sum(ez, axis=1, keepdims=True))
        out_ref[...] = logits - mx - lse


_fin = pl.pallas_call(
    _fin_body,
    grid=(_RB,),
    in_specs=[
        pl.BlockSpec((_NC, 1024, _D), lambda i: (0, i, 0)),
        pl.BlockSpec((1024, 1), lambda i: (i, 0)),
        pl.BlockSpec((1, _D), lambda i: (0, 0)),
        pl.BlockSpec((1, 1024), lambda i: (0, i)),
        pl.BlockSpec((_D, _D), lambda i: (0, 0)),
        pl.BlockSpec((1, _D), lambda i: (0, 0)),
    ],
    out_specs=pl.BlockSpec((_G, _D), lambda i: (0, 0)),
    out_shape=jax.ShapeDtypeStruct((_G, _D), jnp.float32),
    scratch_shapes=[pltpu.VMEM((_G, _D), jnp.float32)],
)


def kernel(x, edge_attr, edge_index, batch, W1, b1, W2, b2, W3, b3, Wc, bc):
    f32 = jnp.float32
    ei = edge_index.astype(jnp.int32)
    loop = jnp.arange(_N, dtype=jnp.int32)
    padi = _N + (jnp.arange(_EPAD - _ETOT, dtype=jnp.int32) % 128)
    srcb = jnp.concatenate([ei[0], loop, padi]).reshape(_NW, _NBLK, _B)
    dstb = jnp.concatenate([ei[1], loop, padi]).reshape(_NW, _NBLK, _B)
    ib = jnp.stack([srcb, dstb], axis=2)
    z1 = jnp.zeros((_NPAD,), f32)
    z2 = jnp.zeros((_STRIPE, _D), f32)
    xp = jnp.pad(x, ((0, _NPAD - _N), (0, 0)))

    degp = _deg_call()(ib, z1).reshape(_NC, _NPAD, 1)

    q, dv = _mm1(xp, W1, degp)
    sp = _prop_call()(q, ib, z2)
    q = _mid(sp, dv, b1.reshape(1, _D), W2)
    sp = _prop_call()(q, ib, z2)
    q = _mid(sp, dv, b2.reshape(1, _D), W3)
    sp = _prop_call()(q, ib, z2)

    bm = jnp.pad(batch.astype(jnp.int32), (0, _NPAD - _N),
                 constant_values=_G).reshape(1, _NPAD)
    wcp = jnp.pad(Wc, ((0, 0), (0, _D - 10)))
    bcp = jnp.pad(bc, (0, _D - 10)).reshape(1, _D)
    outp = _fin(sp, dv, b3.reshape(1, _D), bm, wcp, bcp)
    return outp[:, :10]


# speedup vs baseline: 28.1181x; 1.0082x over previous
import functools

import jax
import jax.numpy as jnp
from jax import lax
from jax.experimental import pallas as pl
from jax.experimental.pallas import tpu as pltpu
from jax.experimental.pallas import tpu_sc as plsc

_N = 10000
_D = 128
_G = 64
_NPAD = 10240
_E = 320000
_ETOT = _E + _N
_B = 128
_NC, _NS = 2, 16
_NW = _NC * _NS
_NBLK = -(-_ETOT // (_B * _NW))
_EPAD = _NBLK * _NW * _B
_STRIPE = _NPAD // _NS
_RB = _NPAD // 1024


def _deg_body(ib, z1, out, ibv, onesv, acc, sdma):
    c = lax.axis_index("c")
    s = lax.axis_index("s")
    w = s * _NC + c
    pltpu.sync_copy(z1.at[pl.ds(s * _STRIPE, _STRIPE)],
                    acc.at[pl.ds(s * _STRIPE, _STRIPE)])
    pltpu.sync_copy(ib.at[w], ibv)
    for g in range(_B // 16):
        onesv[pl.ds(g * 16, 16)] = jnp.ones((16,), jnp.float32)
    plsc.subcore_barrier()

    def step(i, carry):
        pltpu.async_copy(onesv, acc.at[ibv.at[i, 1]], sdma, add=True)
        return carry

    lax.fori_loop(0, _NBLK, step, 0)

    def drain(i, carry):
        pltpu.make_async_copy(onesv, acc.at[ibv.at[i, 1]], sdma).wait()
        return carry

    lax.fori_loop(0, _NBLK, drain, 0)
    plsc.subcore_barrier()
    pltpu.sync_copy(acc.at[pl.ds(s * _STRIPE, _STRIPE)],
                    out.at[pl.ds(c * _NPAD + s * _STRIPE, _STRIPE)])


@functools.cache
def _deg_call():
    mesh = plsc.VectorSubcoreMesh(core_axis_name="c", subcore_axis_name="s",
                                  num_cores=_NC, num_subcores=_NS)
    return pl.kernel(
        _deg_body,
        out_type=jax.ShapeDtypeStruct((_NC * _NPAD,), jnp.float32),
        mesh=mesh,
        scratch_types=[
            pltpu.VMEM((_NBLK, 2, _B), jnp.int32),
            pltpu.VMEM((_B,), jnp.float32),
            pltpu.VMEM_SHARED((_NPAD,), jnp.float32),
            pltpu.SemaphoreType.DMA,
        ],
    )


def _prop_body(q, ib, z2, out, ib0, ib1, ib2, rows0, rows1, acc,
               is0, is1, is2, sem0, sem1):
    c = lax.axis_index("c")
    s = lax.axis_index("s")
    w = s * _NC + c

    def start_i(j, buf, sem):
        pltpu.async_copy(ib.at[w, j], buf, sem)

    def wait_i(buf, sem):
        pltpu.make_async_copy(ib.at[w, 0], buf, sem).wait()

    def start_g(ibuf, buf, sem):
        pltpu.async_copy(q.at[ibuf.at[0]], buf, sem)

    def wait_g(buf, sem):
        pltpu.make_async_copy(q.at[pl.ds(0, _B)], buf, sem).wait()

    def scat(ibuf, buf):
        pltpu.sync_copy(buf, acc.at[ibuf.at[1]], add=True)

    s0, s1, s2 = (ib0, is0), (ib1, is1), (ib2, is2)

    def sub(blk, sA, sB, sC, tail):
        ibA, isA = sA
        ibB, isB = sB
        ibC, isC = sC
        wait_i(ibB, isB)
        start_g(ibB, rows1, sem1)
        wait_g(rows0, sem0)
        scat(ibA, rows0)
        if not tail:
            start_i(blk + 3, ibA, isA)
        wait_i(ibC, isC)
        start_g(ibC, rows0, sem0)
        wait_g(rows1, sem1)
        scat(ibB, rows1)
        if not tail:
            start_i(blk + 4, ibB, isB)

    start_i(0, ib0, is0)
    start_i(1, ib1, is1)
    start_i(2, ib2, is2)
    pltpu.sync_copy(z2, acc.at[pl.ds(s * _STRIPE, _STRIPE)])
    wait_i(ib0, is0)
    start_g(ib0, rows0, sem0)
    plsc.subcore_barrier()

    def step(k, carry):
        b = 6 * k
        sub(b, s0, s1, s2, False)
        sub(b + 2, s2, s0, s1, False)
        sub(b + 4, s1, s2, s0, False)
        return carry

    lax.fori_loop(0, (_NBLK - 3) // 6, step, 0)
    sub(_NBLK - 3, s0, s1, s2, True)
    wait_g(rows0, sem0)
    scat(ib2, rows0)
    plsc.subcore_barrier()
    pltpu.sync_copy(acc.at[pl.ds(s * _STRIPE, _STRIPE)],
                    out.at[c].at[pl.ds(s * _STRIPE, _STRIPE)])


@functools.cache
def _prop_call():
    mesh = plsc.VectorSubcoreMesh(core_axis_name="c", subcore_axis_name="s",
                                  num_cores=_NC, num_subcores=_NS)
    return pl.kernel(
        _prop_body,
        out_type=jax.ShapeDtypeStruct((_NC, _NPAD, _D), jnp.float32),
        mesh=mesh,
        scratch_types=[
            pltpu.VMEM((2, _B), jnp.int32),
            pltpu.VMEM((2, _B), jnp.int32),
            pltpu.VMEM((2, _B), jnp.int32),
            pltpu.VMEM((_B, _D), jnp.float32),
            pltpu.VMEM((_B, _D), jnp.float32),
            pltpu.VMEM_SHARED((_NPAD, _D), jnp.float32),
            pltpu.SemaphoreType.DMA,
            pltpu.SemaphoreType.DMA,
            pltpu.SemaphoreType.DMA,
            pltpu.SemaphoreType.DMA,
            pltpu.SemaphoreType.DMA,
        ],
    )


def _mm1_body(x_ref, w_ref, dg_ref, q_ref, dv_ref):
    d = dg_ref[0] + dg_ref[1]
    dinv = jnp.where(d > 0, lax.rsqrt(d), 0.0)
    dv_ref[...] = dinv
    q_ref[...] = dinv * jnp.dot(x_ref[...], w_ref[...],
                                preferred_element_type=jnp.float32)


_mm1 = pl.pallas_call(
    _mm1_body,
    grid=(_RB,),
    in_specs=[
        pl.BlockSpec((1024, _D), lambda i: (i, 0)),
        pl.BlockSpec((_D, _D), lambda i: (0, 0)),
        pl.BlockSpec((_NC, 1024, 1), lambda i: (0, i, 0)),
    ],
    out_specs=[
        pl.BlockSpec((1024, _D), lambda i: (i, 0)),
        pl.BlockSpec((1024, 1), lambda i: (i, 0)),
    ],
    out_shape=[
        jax.ShapeDtypeStruct((_NPAD, _D), jnp.float32),
        jax.ShapeDtypeStruct((_NPAD, 1), jnp.float32),
    ],
)


def _mid_body(sp_ref, dv_ref, b_ref, w_ref, q_ref):
    dinv = dv_ref[...]
    h = jnp.maximum(dinv * (sp_ref[0] + sp_ref[1]) + b_ref[...], 0.0)
    q_ref[...] = dinv * jnp.dot(h, w_ref[...],
                                preferred_element_type=jnp.float32)


_mid = pl.pallas_call(
    _mid_body,
    grid=(_RB,),
    in_specs=[
        pl.BlockSpec((_NC, 1024, _D), lambda i: (0, i, 0)),
        pl.BlockSpec((1024, 1), lambda i: (i, 0)),
        pl.BlockSpec((1, _D), lambda i: (0, 0)),
        pl.BlockSpec((_D, _D), lambda i: (0, 0)),
    ],
    out_specs=pl.BlockSpec((1024, _D), lambda i: (i, 0)),
    out_shape=jax.ShapeDtypeStruct((_NPAD, _D), jnp.float32),
)


def _fin_body(sp_ref, dv_ref, b_ref, bm_ref, wc_ref, bc_ref, out_ref):
    dinv = dv_ref[...]
    h = jnp.maximum(dinv * (sp_ref[0] + sp_ref[1]) + b_ref[...], 0.0)
    gids = lax.broadcasted_iota(jnp.int32, (_G, _NPAD), 0)
    m = (gids == bm_ref[...]).astype(jnp.float32)
    pooled = jnp.dot(m, h, preferred_element_type=jnp.float32)
    logits = jnp.dot(pooled, wc_ref[...],
                     preferred_element_type=jnp.float32) + bc_ref[...]
    col = lax.broadcasted_iota(jnp.int32, (_G, _D), 1)
    valid = col < 10
    neg = jnp.where(valid, logits, -jnp.inf)
    mx = jnp.max(neg, axis=1, keepdims=True)
    ez = jnp.where(valid, jnp.exp(logits - mx), 0.0)
    lse = jnp.log(jnp.sum(ez, axis=1, keepdims=True))
    out_ref[...] = logits - mx - lse


_fin = pl.pallas_call(
    _fin_body,
    out_shape=jax.ShapeDtypeStruct((_G, _D), jnp.float32),
)


def kernel(x, edge_attr, edge_index, batch, W1, b1, W2, b2, W3, b3, Wc, bc):
    f32 = jnp.float32
    ei = edge_index.astype(jnp.int32)
    loop = jnp.arange(_N, dtype=jnp.int32)
    padi = _N + (jnp.arange(_EPAD - _ETOT, dtype=jnp.int32) % 128)
    srcb = jnp.concatenate([ei[0], loop, padi]).reshape(_NW, _NBLK, _B)
    dstb = jnp.concatenate([ei[1], loop, padi]).reshape(_NW, _NBLK, _B)
    ib = jnp.stack([srcb, dstb], axis=2)
    z1 = jnp.zeros((_NPAD,), f32)
    z2 = jnp.zeros((_STRIPE, _D), f32)
    xp = jnp.pad(x, ((0, _NPAD - _N), (0, 0)))

    degp = _deg_call()(ib, z1).reshape(_NC, _NPAD, 1)

    q, dv = _mm1(xp, W1, degp)
    sp = _prop_call()(q, ib, z2)
    q = _mid(sp, dv, b1.reshape(1, _D), W2)
    sp = _prop_call()(q, ib, z2)
    q = _mid(sp, dv, b2.reshape(1, _D), W3)
    sp = _prop_call()(q, ib, z2)

    bm = jnp.pad(batch.astype(jnp.int32), (0, _NPAD - _N),
                 constant_values=_G).reshape(1, _NPAD)
    wcp = jnp.pad(Wc, ((0, 0), (0, _D - 10)))
    bcp = jnp.pad(bc, (0, _D - 10)).reshape(1, _D)
    outp = _fin(sp, dv, b3.reshape(1, _D), bm, wcp, bcp)
    return outp[:, :10]
